# baseline (speedup 1.0000x reference)
import functools

import jax
import jax.numpy as jnp
from jax import lax
from jax.experimental import pallas as pl
from jax.experimental.pallas import tpu as pltpu
from jax.experimental.pallas import tpu_sc as plsc

_B, _F, _H, _W = 8, 100000, 512, 512
_CH = 2048
_NK = 48
_FA = _NK * _CH
_FT = _F - _FA
_NST = 3 * _NK
_GV = 32
_HASH = 4096
_RC = 4
_RCPX = _RC * _W
_BIG = 1000000.0


def _chunk_base(k):
    return k * _CH


def _raster_body(vt, ct, vtl, ctl, img, tri, dep, CS,
                 xb0, xb1, yb0, yb1, zb0, zb1,
                 dmin, sbuf, hbuf, idxb, gsb0, gsb1, gsb2, cb0, cb1, cb2,
                 planes, tstage, sw0, sw1, s0, s1, sg):
    iota = lax.iota(jnp.int32, 16)
    fzero = iota * jnp.float32(0.0)
    cid = lax.axis_index("c")
    sid_ax = lax.axis_index("s")
    wid = cid * 16 + sid_ax
    b = wid >> 2
    band = wid & 3
    b0 = cid * 4

    def crow_body(tr, _):
        @pl.when((tr & 15) == sid_ax)
        def _do():
            bl = tr & 3
            vc = tr >> 2
            v = vc // 3
            c = vc - v * 3
            bb = b0 + bl
            base = (vc * _B + bb) * _F

            def src(k):
                return ct.at[v, c, bb, pl.ds(_chunk_base(k), _CH)]

            def dst(k):
                return CS.at[pl.ds(base + _chunk_base(k), _CH)]

            pltpu.async_copy(src(0), xb0, sw0)
            pltpu.async_copy(src(1), xb1, sw1)

            def ck_body(u, _):
                k0 = 2 * u
                pltpu.make_async_copy(src(k0), xb0, sw0).wait()
                pltpu.sync_copy(xb0, dst(k0))

                @pl.when(k0 + 2 < _NK)
                def _p0():
                    pltpu.async_copy(src(k0 + 2), xb0, sw0)

                @pl.when(k0 + 1 < _NK)
                def _odd():
                    pltpu.make_async_copy(src(k0 + 1), xb1, sw1).wait()
                    pltpu.sync_copy(xb1, dst(k0 + 1))

                    @pl.when(k0 + 3 < _NK)
                    def _p1():
                        pltpu.async_copy(src(k0 + 3), xb1, sw1)
                return 0
            lax.fori_loop(0, (_NK + 1) // 2, ck_body, 0)
            pltpu.sync_copy(ctl.at[pl.ds((vc * _B + bb) * _FT, _FT)],
                            xb0.at[pl.ds(0, _FT)])
            pltpu.sync_copy(xb0.at[pl.ds(0, _FT)],
                            CS.at[pl.ds(base + _FA, _FT)])
        return 0
    lax.fori_loop(0, 36, crow_body, 0)
    plsc.subcore_barrier()

    def start(t, bufs, sem):
        k = t // 3
        v = t - k * 3

        @pl.when(t < _NST)
        def _main():
            for c, buf in enumerate(bufs):
                pltpu.async_copy(vt.at[v, c, b, pl.ds(k * _CH, _CH)], buf, sem)

        @pl.when(t >= _NST)
        def _tail():
            for c, buf in enumerate(bufs):
                pltpu.async_copy(
                    vtl.at[pl.ds(((v * 3 + c) * _B + b) * _CH, _CH)], buf, sem)

    def wait_for(t, bufs, sem):
        k = t // 3
        v = t - k * 3

        @pl.when(t < _NST)
        def _main():
            for c, buf in enumerate(bufs):
                pltpu.make_async_copy(vt.at[v, c, b, pl.ds(k * _CH, _CH)],
                                      buf, sem).wait()

        @pl.when(t >= _NST)
        def _tail():
            for c, buf in enumerate(bufs):
                pltpu.make_async_copy(
                    vtl.at[pl.ds(((v * 3 + c) * _B + b) * _CH, _CH)],
                    buf, sem).wait()

    def superpass(sp, _):
        bandid = band * 2 + sp
        row0 = bandid * 64

        def init_body(i, _):
            r = i >> 5
            c = (i & 31) * 16
            dmin[r, pl.ds(c, 16)] = fzero + _BIG
            sbuf[r, pl.ds(c, 16)] = iota * 0 - 1
            return 0
        lax.fori_loop(0, 64 * 32, init_body, 0)

        def process(sbase, bufs):
            xb_, yb_, zb_ = bufs

            def decode(off):
                x = xb_[pl.ds(off, 16)]
                y = yb_[pl.ds(off, 16)]
                z = zb_[pl.ds(off, 16)]
                px = (x * 512.0).astype(jnp.int32)
                py = (y * 512.0).astype(jnp.int32)
                m = (py >> 6) == bandid
                rl = py & 63
                sid = sbase + (off + iota) * 3
                return z, px, rl, m, sid

            def rmw(z, px, rl, sid, mask):
                gd = plsc.load_gather(dmin, [rl, px], mask=mask)
                gs = plsc.load_gather(sbuf, [rl, px], mask=mask)
                wm = mask & ((z < gd) | ((z == gd) & (sid > gs)))
                plsc.store_scatter(dmin, [rl, px], z, mask=wm)
                plsc.store_scatter(sbuf, [rl, px], sid, mask=wm)

            def group_body(g, _):
                gbase = g * (_GV * 16)
                bacc = iota < 0
                for i in range(_GV):
                    off = gbase + i * 16
                    z, px, rl, m, sid = decode(off)
                    hv = ((rl & 7) << 9) | px
                    plsc.store_scatter(hbuf, [hv], iota, mask=m)
                    gl = plsc.load_gather(hbuf, [hv], mask=m)
                    bacc = bacc | (m & (gl != iota))
                    rmw(z, px, rl, sid, m & (gl == iota))
                anybad = jnp.max(jnp.where(bacc, 1, 0))

                @pl.when(anybad > 0)
                def _slow():
                    def sl_body(q, _):
                        off = gbase + (q >> 4) * 16
                        z, px, rl, m, sid = decode(off)
                        rmw(z, px, rl, sid, m & (iota == (q & 15)))
                        return 0
                    lax.fori_loop(0, _GV * 16, sl_body, 0)
                return 0
            lax.fori_loop(0, (_CH // 16) // _GV, group_body, 0, unroll=2)

        bufs0 = (xb0, yb0, zb0)
        bufs1 = (xb1, yb1, zb1)
        start(0, bufs0, s0)
        start(1, bufs1, s1)

        def sbase_of(t):
            k = t // 3
            v = t - k * 3
            return _chunk_base(k) * 3 + v

        NT = _NST + 3

        def chunk_body(u, _):
            t0 = 2 * u
            wait_for(t0, bufs0, s0)
            process(sbase_of(t0), bufs0)

            @pl.when(t0 + 2 < NT)
            def _pf0():
                start(t0 + 2, bufs0, s0)

            @pl.when(t0 + 1 < NT)
            def _odd():
                wait_for(t0 + 1, bufs1, s1)
                process(sbase_of(t0 + 1), bufs1)

                @pl.when(t0 + 3 < NT)
                def _pf1():
                    start(t0 + 3, bufs1, s1)
            return 0
        lax.fori_loop(0, (NT + 1) // 2, chunk_body, 0)

        pltpu.sync_copy(dmin, dep.at[b, pl.ds(row0, 64), :])

        def rc_body(rc, _):
            def cv_body(v_, cnt):
                r = v_ >> 5
                c = (v_ & 31) * 16
                sb = sbuf[rc * _RC + r, pl.ds(c, 16)]
                cov = sb >= 0
                fid = sb // 3
                tstage[r, pl.ds(c, 16)] = jnp.where(cov, fid, -1)
                vtx = sb - fid * 3
                g0 = (vtx * 3 * _B + b) * _F + fid
                pixv = r * 512 + c + iota
                plsc.store_compressed(idxb.at[pl.ds(cnt, 16)], pixv, mask=cov)
                plsc.store_compressed(gsb0.at[pl.ds(cnt, 16)], g0, mask=cov)
                plsc.store_compressed(gsb1.at[pl.ds(cnt, 16)], g0 + _B * _F, mask=cov)
                plsc.store_compressed(gsb2.at[pl.ds(cnt, 16)], g0 + 2 * _B * _F, mask=cov)
                pc = plsc.all_reduce_population_count(cov)
                return cnt + jnp.max(pc)
            cnt = lax.fori_loop(0, _RCPX // 16, cv_body, jnp.int32(0))

            pltpu.sync_copy(tstage, tri.at[b, pl.ds(row0 + rc * _RC, _RC), :])

            for pv in range(8):
                pad = wid * 128 + pv * 16 + iota
                gsb0[pl.ds(cnt + pv * 16, 16)] = pad
                gsb1[pl.ds(cnt + pv * 16, 16)] = pad + _B * _F
                gsb2[pl.ds(cnt + pv * 16, 16)] = pad + 2 * _B * _F

            ng = (cnt + 127) >> 7

            def fire(g, _):
                for gsb_, cb_ in ((gsb0, cb0), (gsb1, cb1), (gsb2, cb2)):
                    pltpu.async_copy(CS.at[gsb_.at[pl.ds(g * 128, 128)]],
                                     cb_.at[pl.ds(g * 128, 128)], sg)
                return 0
            lax.fori_loop(0, ng, fire, 0)

            def zero_body(i, _):
                r = i >> 5
                c = (i & 31) * 16
                planes[0, r, pl.ds(c, 16)] = fzero
                planes[1, r, pl.ds(c, 16)] = fzero
                planes[2, r, pl.ds(c, 16)] = fzero
                return 0
            lax.fori_loop(0, _RCPX // 16, zero_body, 0)

            def drain(g, _):
                for gsb_, cb_ in ((gsb0, cb0), (gsb1, cb1), (gsb2, cb2)):
                    pltpu.make_async_copy(CS.at[gsb_.at[pl.ds(g * 128, 128)]],
                                          cb_.at[pl.ds(g * 128, 128)], sg).wait()
                return 0
            lax.fori_loop(0, ng, drain, 0)

            def sc_body(vw, _):
                pos = vw * 16
                am = (pos + iota) < cnt
                lp = idxb[pl.ds(pos, 16)]
                pr = lp >> 9
                pc_ = lp & 511
                for ch, cb_ in enumerate((cb0, cb1, cb2)):
                    cvv = cb_[pl.ds(pos, 16)]
                    plsc.store_scatter(planes, [iota * 0 + ch, pr, pc_], cvv, mask=am)
                return 0
            lax.fori_loop(0, (cnt + 15) >> 4, sc_body, 0)

            for ch in range(3):
                pltpu.sync_copy(planes.at[ch],
                                img.at[b, ch, pl.ds(row0 + rc * _RC, _RC), :])
            return 0
        lax.fori_loop(0, 64 // _RC, rc_body, 0)
        return 0

    lax.fori_loop(0, 2, superpass, 0)


@functools.partial(
    pl.kernel,
    out_type=(
        jax.ShapeDtypeStruct((_B, 3, _H, _W), jnp.float32),
        jax.ShapeDtypeStruct((_B, _H, _W), jnp.int32),
        jax.ShapeDtypeStruct((_B, _H, _W), jnp.float32),
        jax.ShapeDtypeStruct((9 * _B * _F,), jnp.float32),
    ),
    mesh=plsc.VectorSubcoreMesh(core_axis_name="c", subcore_axis_name="s"),
    scratch_types=[
        pltpu.VMEM((_CH,), jnp.float32),
        pltpu.VMEM((_CH,), jnp.float32),
        pltpu.VMEM((_CH,), jnp.float32),
        pltpu.VMEM((_CH,), jnp.float32),
        pltpu.VMEM((_CH,), jnp.float32),
        pltpu.VMEM((_CH,), jnp.float32),
        pltpu.VMEM((64, _W), jnp.float32),
        pltpu.VMEM((64, _W), jnp.int32),
        pltpu.VMEM((_HASH,), jnp.int32),
        pltpu.VMEM((_RCPX + 128,), jnp.int32),
        pltpu.VMEM((_RCPX + 128,), jnp.int32),
        pltpu.VMEM((_RCPX + 128,), jnp.int32),
        pltpu.VMEM((_RCPX + 128,), jnp.int32),
        pltpu.VMEM((_RCPX + 128,), jnp.float32),
        pltpu.VMEM((_RCPX + 128,), jnp.float32),
        pltpu.VMEM((_RCPX + 128,), jnp.float32),
        pltpu.VMEM((3, _RC, _W), jnp.float32),
        pltpu.VMEM((_RC, _W), jnp.int32),
        pltpu.SemaphoreType.DMA,
        pltpu.SemaphoreType.DMA,
        pltpu.SemaphoreType.DMA,
        pltpu.SemaphoreType.DMA,
        pltpu.SemaphoreType.DMA,
    ],
    compiler_params=pltpu.CompilerParams(needs_layout_passes=False),
)
def _raster(vt, ct, vtl, ctl, img, tri, dep, CS, *scratch):
    _raster_body(vt, ct, vtl, ctl, img, tri, dep, CS, *scratch)


def kernel(face_vertices, face_colors, return_buffers):
    vt = jnp.transpose(face_vertices, (2, 3, 0, 1))
    ct = jnp.transpose(face_colors, (2, 3, 0, 1))
    vtail = vt[:, :, :, _FA:]
    fi = jnp.arange(_FT, _CH, dtype=jnp.float32)
    xp = jnp.broadcast_to(((fi % 512.0) + 0.5) / 512.0, (3, _B, _CH - _FT))
    yp = jnp.zeros((3, _B, _CH - _FT), jnp.float32) + (0.5 / 512.0)
    zp = jnp.zeros((3, _B, _CH - _FT), jnp.float32) + 2.0 * _BIG
    pad = jnp.stack([xp, yp, zp], axis=1)
    vtl = jnp.concatenate([vtail, pad], axis=3).reshape(-1)
    ctl = ct[:, :, :, _FA:].reshape(-1)
    images, tri, depth, _ = _raster(vt, ct, vtl, ctl)
    flag = jnp.asarray(return_buffers)
    return lax.cond(
        flag,
        lambda: (images, tri, depth),
        lambda: (jnp.zeros_like(images), jnp.full_like(tri, -1),
                 jnp.full_like(depth, _BIG)),
    )


# speedup vs baseline: 1.6280x; 1.6280x over previous
import functools

import jax
import jax.numpy as jnp
from jax import lax
from jax.experimental import pallas as pl
from jax.experimental.pallas import tpu as pltpu
from jax.experimental.pallas import tpu_sc as plsc

_B, _F, _H, _W = 8, 100000, 512, 512
_CH = 2048
_NK = 48
_FA = _NK * _CH
_FT = _F - _FA
_NST = 3 * _NK
_GV = 32
_HASH = 4096
_RC = 4
_RCPX = _RC * _W
_BIG = 1000000.0


def _chunk_base(k):
    return k * _CH


def _raster_body(vt, ct, vtl, ctl, img, tri, dep, CS,
                 xb0, xb1, yb0, yb1, zb0, zb1,
                 dmin, sbuf, hbuf, idxb, gsb0, gsb1, gsb2, cb0, cb1, cb2,
                 planes, tstage, so, sw0, sw1, s0, s1, sg):
    iota = lax.iota(jnp.int32, 16)
    fzero = iota * jnp.float32(0.0)
    cid = lax.axis_index("c")
    sid_ax = lax.axis_index("s")
    wid = cid * 16 + sid_ax
    b = wid >> 2
    band = wid & 3
    b0 = cid * 4

    def crow_body(tr, _):
        @pl.when((tr & 15) == sid_ax)
        def _do():
            bl = tr & 3
            vc = tr >> 2
            v = vc // 3
            c = vc - v * 3
            bb = b0 + bl
            base = (vc * _B + bb) * _F

            def src(k):
                return ct.at[v, c, bb, pl.ds(_chunk_base(k), _CH)]

            def dst(k):
                return CS.at[pl.ds(base + _chunk_base(k), _CH)]

            pltpu.async_copy(src(0), xb0, sw0)
            pltpu.async_copy(src(1), xb1, sw1)

            def ck_body(u, _):
                k0 = 2 * u
                pltpu.make_async_copy(src(k0), xb0, sw0).wait()
                pltpu.sync_copy(xb0, dst(k0))

                @pl.when(k0 + 2 < _NK)
                def _p0():
                    pltpu.async_copy(src(k0 + 2), xb0, sw0)

                @pl.when(k0 + 1 < _NK)
                def _odd():
                    pltpu.make_async_copy(src(k0 + 1), xb1, sw1).wait()
                    pltpu.sync_copy(xb1, dst(k0 + 1))

                    @pl.when(k0 + 3 < _NK)
                    def _p1():
                        pltpu.async_copy(src(k0 + 3), xb1, sw1)
                return 0
            lax.fori_loop(0, (_NK + 1) // 2, ck_body, 0)
            pltpu.sync_copy(ctl.at[pl.ds((vc * _B + bb) * _FT, _FT)],
                            xb0.at[pl.ds(0, _FT)])
            pltpu.sync_copy(xb0.at[pl.ds(0, _FT)],
                            CS.at[pl.ds(base + _FA, _FT)])
        return 0
    lax.fori_loop(0, 36, crow_body, 0)
    plsc.subcore_barrier()

    def start(t, bufs, sem):
        k = t // 3
        v = t - k * 3

        @pl.when(t < _NST)
        def _main():
            for c, buf in enumerate(bufs):
                pltpu.async_copy(vt.at[v, c, b, pl.ds(k * _CH, _CH)], buf, sem)

        @pl.when(t >= _NST)
        def _tail():
            for c, buf in enumerate(bufs):
                pltpu.async_copy(
                    vtl.at[pl.ds(((v * 3 + c) * _B + b) * _CH, _CH)], buf, sem)

    def wait_for(t, bufs, sem):
        k = t // 3
        v = t - k * 3

        @pl.when(t < _NST)
        def _main():
            for c, buf in enumerate(bufs):
                pltpu.make_async_copy(vt.at[v, c, b, pl.ds(k * _CH, _CH)],
                                      buf, sem).wait()

        @pl.when(t >= _NST)
        def _tail():
            for c, buf in enumerate(bufs):
                pltpu.make_async_copy(
                    vtl.at[pl.ds(((v * 3 + c) * _B + b) * _CH, _CH)],
                    buf, sem).wait()

    def superpass(sp, _):
        bandid = band * 2 + sp
        row0 = bandid * 64

        def init_body(i, _):
            r = i >> 5
            c = (i & 31) * 16
            dmin[r, pl.ds(c, 16)] = fzero + _BIG
            sbuf[r, pl.ds(c, 16)] = iota * 0 - 1
            return 0
        lax.fori_loop(0, 64 * 32, init_body, 0)

        def process(sbase, bufs):
            xb_, yb_, zb_ = bufs

            def decode(off):
                x = xb_[pl.ds(off, 16)]
                y = yb_[pl.ds(off, 16)]
                z = zb_[pl.ds(off, 16)]
                px = (x * 512.0).astype(jnp.int32)
                py = (y * 512.0).astype(jnp.int32)
                m = (py >> 6) == bandid
                rl = py & 63
                sid = sbase + (off + iota) * 3
                return z, px, rl, m, sid

            def rmw(z, px, rl, sid, mask):
                gd = plsc.load_gather(dmin, [rl, px], mask=mask)
                gs = plsc.load_gather(sbuf, [rl, px], mask=mask)
                wm = mask & ((z < gd) | ((z == gd) & (sid > gs)))
                plsc.store_scatter(dmin, [rl, px], z, mask=wm)
                plsc.store_scatter(sbuf, [rl, px], sid, mask=wm)

            def group_body(g, _):
                gbase = g * (_GV * 16)
                bacc = iota < 0
                for i in range(_GV):
                    off = gbase + i * 16
                    z, px, rl, m, sid = decode(off)
                    hv = ((rl & 7) << 9) | px
                    plsc.store_scatter(hbuf, [hv], iota, mask=m)
                    gl = plsc.load_gather(hbuf, [hv], mask=m)
                    bacc = bacc | (m & (gl != iota))
                    rmw(z, px, rl, sid, m & (gl == iota))
                anybad = jnp.max(jnp.where(bacc, 1, 0))

                @pl.when(anybad > 0)
                def _slow():
                    def sl_body(q, _):
                        off = gbase + (q >> 4) * 16
                        z, px, rl, m, sid = decode(off)
                        rmw(z, px, rl, sid, m & (iota == (q & 15)))
                        return 0
                    lax.fori_loop(0, _GV * 16, sl_body, 0)
                return 0
            lax.fori_loop(0, (_CH // 16) // _GV, group_body, 0)

        bufs0 = (xb0, yb0, zb0)
        bufs1 = (xb1, yb1, zb1)
        start(0, bufs0, s0)
        start(1, bufs1, s1)

        def sbase_of(t):
            k = t // 3
            v = t - k * 3
            return _chunk_base(k) * 3 + v

        NT = _NST + 3

        def chunk_body(u, _):
            t0 = 2 * u
            wait_for(t0, bufs0, s0)
            process(sbase_of(t0), bufs0)

            @pl.when(t0 + 2 < NT)
            def _pf0():
                start(t0 + 2, bufs0, s0)

            @pl.when(t0 + 1 < NT)
            def _odd():
                wait_for(t0 + 1, bufs1, s1)
                process(sbase_of(t0 + 1), bufs1)

                @pl.when(t0 + 3 < NT)
                def _pf1():
                    start(t0 + 3, bufs1, s1)
            return 0
        lax.fori_loop(0, (NT + 1) // 2, chunk_body, 0)

        pltpu.sync_copy(dmin, dep.at[b, pl.ds(row0, 64), :])

        def drain_out(rcp):
            par = rcp & 1
            pltpu.make_async_copy(tstage.at[par],
                                  tri.at[b, pl.ds(row0 + rcp * _RC, _RC), :],
                                  so).wait()
            for ch in range(3):
                pltpu.make_async_copy(
                    planes.at[par, ch],
                    img.at[b, ch, pl.ds(row0 + rcp * _RC, _RC), :], so).wait()

        def rc_body(rc, _):
            par = rc & 1

            @pl.when(rc > 1)
            def _drain_prev():
                drain_out(rc - 2)

            def cv_body(v_, cnt):
                r = v_ >> 5
                c = (v_ & 31) * 16
                sb = sbuf[rc * _RC + r, pl.ds(c, 16)]
                cov = sb >= 0
                fid = sb // 3
                tstage[par, r, pl.ds(c, 16)] = jnp.where(cov, fid, -1)
                vtx = sb - fid * 3
                g0 = (vtx * 3 * _B + b) * _F + fid
                pixv = r * 512 + c + iota
                plsc.store_compressed(idxb.at[pl.ds(cnt, 16)], pixv, mask=cov)
                plsc.store_compressed(gsb0.at[pl.ds(cnt, 16)], g0, mask=cov)
                plsc.store_compressed(gsb1.at[pl.ds(cnt, 16)], g0 + _B * _F, mask=cov)
                plsc.store_compressed(gsb2.at[pl.ds(cnt, 16)], g0 + 2 * _B * _F, mask=cov)
                pc = plsc.all_reduce_population_count(cov)
                return cnt + jnp.max(pc)
            cnt = lax.fori_loop(0, _RCPX // 16, cv_body, jnp.int32(0))

            pltpu.async_copy(tstage.at[par],
                             tri.at[b, pl.ds(row0 + rc * _RC, _RC), :], so)

            for pv in range(8):
                pad = wid * 128 + pv * 16 + iota
                gsb0[pl.ds(cnt + pv * 16, 16)] = pad
                gsb1[pl.ds(cnt + pv * 16, 16)] = pad + _B * _F
                gsb2[pl.ds(cnt + pv * 16, 16)] = pad + 2 * _B * _F

            ng = (cnt + 127) >> 7

            def fire(g, _):
                for gsb_, cb_ in ((gsb0, cb0), (gsb1, cb1), (gsb2, cb2)):
                    pltpu.async_copy(CS.at[gsb_.at[pl.ds(g * 128, 128)]],
                                     cb_.at[pl.ds(g * 128, 128)], sg)
                return 0
            lax.fori_loop(0, ng, fire, 0)

            def zero_body(i, _):
                r = i >> 5
                c = (i & 31) * 16
                planes[par, 0, r, pl.ds(c, 16)] = fzero
                planes[par, 1, r, pl.ds(c, 16)] = fzero
                planes[par, 2, r, pl.ds(c, 16)] = fzero
                return 0
            lax.fori_loop(0, _RCPX // 16, zero_body, 0)

            def drain(g, _):
                for gsb_, cb_ in ((gsb0, cb0), (gsb1, cb1), (gsb2, cb2)):
                    pltpu.make_async_copy(CS.at[gsb_.at[pl.ds(g * 128, 128)]],
                                          cb_.at[pl.ds(g * 128, 128)], sg).wait()
                return 0
            lax.fori_loop(0, ng, drain, 0)

            def sc_body(vw, _):
                pos = vw * 16
                am = (pos + iota) < cnt
                lp = idxb[pl.ds(pos, 16)]
                pr = lp >> 9
                pc_ = lp & 511
                for ch, cb_ in enumerate((cb0, cb1, cb2)):
                    cvv = cb_[pl.ds(pos, 16)]
                    plsc.store_scatter(planes, [iota * 0 + par, iota * 0 + ch, pr, pc_],
                                       cvv, mask=am)
                return 0
            lax.fori_loop(0, (cnt + 15) >> 4, sc_body, 0)

            for ch in range(3):
                pltpu.async_copy(planes.at[par, ch],
                                 img.at[b, ch, pl.ds(row0 + rc * _RC, _RC), :], so)
            return 0
        lax.fori_loop(0, 64 // _RC, rc_body, 0)
        drain_out(64 // _RC - 2)
        drain_out(64 // _RC - 1)
        return 0

    lax.fori_loop(0, 2, superpass, 0)


@functools.partial(
    pl.kernel,
    out_type=(
        jax.ShapeDtypeStruct((_B, 3, _H, _W), jnp.float32),
        jax.ShapeDtypeStruct((_B, _H, _W), jnp.int32),
        jax.ShapeDtypeStruct((_B, _H, _W), jnp.float32),
        jax.ShapeDtypeStruct((9 * _B * _F,), jnp.float32),
    ),
    mesh=plsc.VectorSubcoreMesh(core_axis_name="c", subcore_axis_name="s"),
    scratch_types=[
        pltpu.VMEM((_CH,), jnp.float32),
        pltpu.VMEM((_CH,), jnp.float32),
        pltpu.VMEM((_CH,), jnp.float32),
        pltpu.VMEM((_CH,), jnp.float32),
        pltpu.VMEM((_CH,), jnp.float32),
        pltpu.VMEM((_CH,), jnp.float32),
        pltpu.VMEM((64, _W), jnp.float32),
        pltpu.VMEM((64, _W), jnp.int32),
        pltpu.VMEM((_HASH,), jnp.int32),
        pltpu.VMEM((_RCPX + 128,), jnp.int32),
        pltpu.VMEM((_RCPX + 128,), jnp.int32),
        pltpu.VMEM((_RCPX + 128,), jnp.int32),
        pltpu.VMEM((_RCPX + 128,), jnp.int32),
        pltpu.VMEM((_RCPX + 128,), jnp.float32),
        pltpu.VMEM((_RCPX + 128,), jnp.float32),
        pltpu.VMEM((_RCPX + 128,), jnp.float32),
        pltpu.VMEM((2, 3, _RC, _W), jnp.float32),
        pltpu.VMEM((2, _RC, _W), jnp.int32),
        pltpu.SemaphoreType.DMA,
        pltpu.SemaphoreType.DMA,
        pltpu.SemaphoreType.DMA,
        pltpu.SemaphoreType.DMA,
        pltpu.SemaphoreType.DMA,
        pltpu.SemaphoreType.DMA,
    ],
    compiler_params=pltpu.CompilerParams(needs_layout_passes=False),
)
def _raster(vt, ct, vtl, ctl, img, tri, dep, CS, *scratch):
    _raster_body(vt, ct, vtl, ctl, img, tri, dep, CS, *scratch)


def kernel(face_vertices, face_colors, return_buffers):
    vt = jnp.transpose(face_vertices, (2, 3, 0, 1))
    ct = jnp.transpose(face_colors, (2, 3, 0, 1))
    vtail = vt[:, :, :, _FA:]
    fi = jnp.arange(_FT, _CH, dtype=jnp.float32)
    xp = jnp.broadcast_to(((fi % 512.0) + 0.5) / 512.0, (3, _B, _CH - _FT))
    yp = jnp.zeros((3, _B, _CH - _FT), jnp.float32) + (0.5 / 512.0)
    zp = jnp.zeros((3, _B, _CH - _FT), jnp.float32) + 2.0 * _BIG
    pad = jnp.stack([xp, yp, zp], axis=1)
    vtl = jnp.concatenate([vtail, pad], axis=3).reshape(-1)
    ctl = ct[:, :, :, _FA:].reshape(-1)
    images, tri, depth, _ = _raster(vt, ct, vtl, ctl)
    flag = jnp.asarray(return_buffers)
    return lax.cond(
        flag,
        lambda: (images, tri, depth),
        lambda: (jnp.zeros_like(images), jnp.full_like(tri, -1),
                 jnp.full_like(depth, _BIG)),
    )


# speedup vs baseline: 1.6692x; 1.0253x over previous
import functools

import jax
import jax.numpy as jnp
from jax import lax
from jax.experimental import pallas as pl
from jax.experimental.pallas import tpu as pltpu
from jax.experimental.pallas import tpu_sc as plsc

_B, _F, _H, _W = 8, 100000, 512, 512
_CH = 4096
_NK = 24
_FA = _NK * _CH
_FT = _F - _FA
_NST = 3 * _NK
_GV = 32
_HASH = 4096
_RC = 4
_RCPX = _RC * _W
_BIG = 1000000.0


def _chunk_base(k):
    return k * _CH


def _raster_body(vt, ct, vtl, ctl, img, tri, dep, CS,
                 xb0, xb1, yb0, yb1, zb0, zb1,
                 dmin, sbuf, hbuf, idxb, gsb0, gsb1, gsb2, cb0, cb1, cb2,
                 planes, tstage, so, sw0, sw1, s0, s1, sg):
    iota = lax.iota(jnp.int32, 16)
    fzero = iota * jnp.float32(0.0)
    cid = lax.axis_index("c")
    sid_ax = lax.axis_index("s")
    wid = cid * 16 + sid_ax
    b = wid >> 2
    band = wid & 3
    b0 = cid * 4

    def crow_body(tr, _):
        @pl.when((tr & 15) == sid_ax)
        def _do():
            bl = tr & 3
            vc = tr >> 2
            v = vc // 3
            c = vc - v * 3
            bb = b0 + bl
            base = (vc * _B + bb) * _F

            def src(k):
                return ct.at[v, c, bb, pl.ds(_chunk_base(k), _CH)]

            def dst(k):
                return CS.at[pl.ds(base + _chunk_base(k), _CH)]

            pltpu.async_copy(src(0), xb0, sw0)
            pltpu.async_copy(src(1), xb1, sw1)

            def ck_body(u, _):
                k0 = 2 * u
                pltpu.make_async_copy(src(k0), xb0, sw0).wait()
                pltpu.sync_copy(xb0, dst(k0))

                @pl.when(k0 + 2 < _NK)
                def _p0():
                    pltpu.async_copy(src(k0 + 2), xb0, sw0)

                @pl.when(k0 + 1 < _NK)
                def _odd():
                    pltpu.make_async_copy(src(k0 + 1), xb1, sw1).wait()
                    pltpu.sync_copy(xb1, dst(k0 + 1))

                    @pl.when(k0 + 3 < _NK)
                    def _p1():
                        pltpu.async_copy(src(k0 + 3), xb1, sw1)
                return 0
            lax.fori_loop(0, (_NK + 1) // 2, ck_body, 0)
            pltpu.sync_copy(ctl.at[pl.ds((vc * _B + bb) * _FT, _FT)],
                            xb0.at[pl.ds(0, _FT)])
            pltpu.sync_copy(xb0.at[pl.ds(0, _FT)],
                            CS.at[pl.ds(base + _FA, _FT)])
        return 0
    lax.fori_loop(0, 36, crow_body, 0)
    plsc.subcore_barrier()

    def start(t, bufs, sem):
        k = t // 3
        v = t - k * 3

        @pl.when(t < _NST)
        def _main():
            for c, buf in enumerate(bufs):
                pltpu.async_copy(vt.at[v, c, b, pl.ds(k * _CH, _CH)], buf, sem)

        @pl.when(t >= _NST)
        def _tail():
            for c, buf in enumerate(bufs):
                pltpu.async_copy(
                    vtl.at[pl.ds(((v * 3 + c) * _B + b) * _CH, _CH)], buf, sem)

    def wait_for(t, bufs, sem):
        k = t // 3
        v = t - k * 3

        @pl.when(t < _NST)
        def _main():
            for c, buf in enumerate(bufs):
                pltpu.make_async_copy(vt.at[v, c, b, pl.ds(k * _CH, _CH)],
                                      buf, sem).wait()

        @pl.when(t >= _NST)
        def _tail():
            for c, buf in enumerate(bufs):
                pltpu.make_async_copy(
                    vtl.at[pl.ds(((v * 3 + c) * _B + b) * _CH, _CH)],
                    buf, sem).wait()

    def superpass(sp, _):
        bandid = band * 2 + sp
        row0 = bandid * 64

        def init_body(i, _):
            r = i >> 5
            c = (i & 31) * 16
            dmin[r, pl.ds(c, 16)] = fzero + _BIG
            sbuf[r, pl.ds(c, 16)] = iota * 0 - 1
            return 0
        lax.fori_loop(0, 64 * 32, init_body, 0)

        def process(sbase, bufs):
            xb_, yb_, zb_ = bufs

            def decode(off):
                x = xb_[pl.ds(off, 16)]
                y = yb_[pl.ds(off, 16)]
                z = zb_[pl.ds(off, 16)]
                px = (x * 512.0).astype(jnp.int32)
                py = (y * 512.0).astype(jnp.int32)
                m = (py >> 6) == bandid
                rl = py & 63
                sid = sbase + (off + iota) * 3
                return z, px, rl, m, sid

            def rmw(z, px, rl, sid, mask):
                gd = plsc.load_gather(dmin, [rl, px], mask=mask)
                gs = plsc.load_gather(sbuf, [rl, px], mask=mask)
                wm = mask & ((z < gd) | ((z == gd) & (sid > gs)))
                plsc.store_scatter(dmin, [rl, px], z, mask=wm)
                plsc.store_scatter(sbuf, [rl, px], sid, mask=wm)

            def group_body(g, _):
                gbase = g * (_GV * 16)
                bacc = iota < 0
                for i in range(_GV):
                    off = gbase + i * 16
                    z, px, rl, m, sid = decode(off)
                    hv = ((rl & 7) << 9) | px
                    plsc.store_scatter(hbuf, [hv], iota, mask=m)
                    gl = plsc.load_gather(hbuf, [hv], mask=m)
                    bacc = bacc | (m & (gl != iota))
                    rmw(z, px, rl, sid, m & (gl == iota))
                anybad = jnp.max(jnp.where(bacc, 1, 0))

                @pl.when(anybad > 0)
                def _slow():
                    def sl_body(q, _):
                        off = gbase + (q >> 4) * 16
                        z, px, rl, m, sid = decode(off)
                        rmw(z, px, rl, sid, m & (iota == (q & 15)))
                        return 0
                    lax.fori_loop(0, _GV * 16, sl_body, 0)
                return 0
            lax.fori_loop(0, (_CH // 16) // _GV, group_body, 0)

        bufs0 = (xb0, yb0, zb0)
        bufs1 = (xb1, yb1, zb1)
        start(0, bufs0, s0)
        start(1, bufs1, s1)

        def sbase_of(t):
            k = t // 3
            v = t - k * 3
            return _chunk_base(k) * 3 + v

        NT = _NST + 3

        def chunk_body(u, _):
            t0 = 2 * u
            wait_for(t0, bufs0, s0)
            process(sbase_of(t0), bufs0)

            @pl.when(t0 + 2 < NT)
            def _pf0():
                start(t0 + 2, bufs0, s0)

            @pl.when(t0 + 1 < NT)
            def _odd():
                wait_for(t0 + 1, bufs1, s1)
                process(sbase_of(t0 + 1), bufs1)

                @pl.when(t0 + 3 < NT)
                def _pf1():
                    start(t0 + 3, bufs1, s1)
            return 0
        lax.fori_loop(0, (NT + 1) // 2, chunk_body, 0)

        pltpu.sync_copy(dmin, dep.at[b, pl.ds(row0, 64), :])

        def drain_out(rcp):
            par = rcp & 1
            pltpu.make_async_copy(tstage.at[par],
                                  tri.at[b, pl.ds(row0 + rcp * _RC, _RC), :],
                                  so).wait()
            for ch in range(3):
                pltpu.make_async_copy(
                    planes.at[par, ch],
                    img.at[b, ch, pl.ds(row0 + rcp * _RC, _RC), :], so).wait()

        def rc_body(rc, _):
            par = rc & 1

            @pl.when(rc > 1)
            def _drain_prev():
                drain_out(rc - 2)

            def cv_body(v_, cnt):
                r = v_ >> 5
                c = (v_ & 31) * 16
                sb = sbuf[rc * _RC + r, pl.ds(c, 16)]
                cov = sb >= 0
                fid = sb // 3
                tstage[par, r, pl.ds(c, 16)] = jnp.where(cov, fid, -1)
                vtx = sb - fid * 3
                g0 = (vtx * 3 * _B + b) * _F + fid
                pixv = r * 512 + c + iota
                plsc.store_compressed(idxb.at[pl.ds(cnt, 16)], pixv, mask=cov)
                plsc.store_compressed(gsb0.at[pl.ds(cnt, 16)], g0, mask=cov)
                plsc.store_compressed(gsb1.at[pl.ds(cnt, 16)], g0 + _B * _F, mask=cov)
                plsc.store_compressed(gsb2.at[pl.ds(cnt, 16)], g0 + 2 * _B * _F, mask=cov)
                pc = plsc.all_reduce_population_count(cov)
                return cnt + jnp.max(pc)
            cnt = lax.fori_loop(0, _RCPX // 16, cv_body, jnp.int32(0))

            pltpu.async_copy(tstage.at[par],
                             tri.at[b, pl.ds(row0 + rc * _RC, _RC), :], so)

            for pv in range(8):
                pad = wid * 128 + pv * 16 + iota
                gsb0[pl.ds(cnt + pv * 16, 16)] = pad
                gsb1[pl.ds(cnt + pv * 16, 16)] = pad + _B * _F
                gsb2[pl.ds(cnt + pv * 16, 16)] = pad + 2 * _B * _F

            ng = (cnt + 127) >> 7

            def fire(g, _):
                for gsb_, cb_ in ((gsb0, cb0), (gsb1, cb1), (gsb2, cb2)):
                    pltpu.async_copy(CS.at[gsb_.at[pl.ds(g * 128, 128)]],
                                     cb_.at[pl.ds(g * 128, 128)], sg)
                return 0
            lax.fori_loop(0, ng, fire, 0)

            def zero_body(i, _):
                r = i >> 5
                c = (i & 31) * 16
                planes[par, 0, r, pl.ds(c, 16)] = fzero
                planes[par, 1, r, pl.ds(c, 16)] = fzero
                planes[par, 2, r, pl.ds(c, 16)] = fzero
                return 0
            lax.fori_loop(0, _RCPX // 16, zero_body, 0)

            def drain(g, _):
                for gsb_, cb_ in ((gsb0, cb0), (gsb1, cb1), (gsb2, cb2)):
                    pltpu.make_async_copy(CS.at[gsb_.at[pl.ds(g * 128, 128)]],
                                          cb_.at[pl.ds(g * 128, 128)], sg).wait()
                return 0
            lax.fori_loop(0, ng, drain, 0)

            def sc_body(vw, _):
                pos = vw * 16
                am = (pos + iota) < cnt
                lp = idxb[pl.ds(pos, 16)]
                pr = lp >> 9
                pc_ = lp & 511
                for ch, cb_ in enumerate((cb0, cb1, cb2)):
                    cvv = cb_[pl.ds(pos, 16)]
                    plsc.store_scatter(planes, [iota * 0 + par, iota * 0 + ch, pr, pc_],
                                       cvv, mask=am)
                return 0
            lax.fori_loop(0, (cnt + 15) >> 4, sc_body, 0)

            for ch in range(3):
                pltpu.async_copy(planes.at[par, ch],
                                 img.at[b, ch, pl.ds(row0 + rc * _RC, _RC), :], so)
            return 0
        lax.fori_loop(0, 64 // _RC, rc_body, 0)
        drain_out(64 // _RC - 2)
        drain_out(64 // _RC - 1)
        return 0

    lax.fori_loop(0, 2, superpass, 0)


@functools.partial(
    pl.kernel,
    out_type=(
        jax.ShapeDtypeStruct((_B, 3, _H, _W), jnp.float32),
        jax.ShapeDtypeStruct((_B, _H, _W), jnp.int32),
        jax.ShapeDtypeStruct((_B, _H, _W), jnp.float32),
        jax.ShapeDtypeStruct((9 * _B * _F,), jnp.float32),
    ),
    mesh=plsc.VectorSubcoreMesh(core_axis_name="c", subcore_axis_name="s"),
    scratch_types=[
        pltpu.VMEM((_CH,), jnp.float32),
        pltpu.VMEM((_CH,), jnp.float32),
        pltpu.VMEM((_CH,), jnp.float32),
        pltpu.VMEM((_CH,), jnp.float32),
        pltpu.VMEM((_CH,), jnp.float32),
        pltpu.VMEM((_CH,), jnp.float32),
        pltpu.VMEM((64, _W), jnp.float32),
        pltpu.VMEM((64, _W), jnp.int32),
        pltpu.VMEM((_HASH,), jnp.int32),
        pltpu.VMEM((_RCPX + 128,), jnp.int32),
        pltpu.VMEM((_RCPX + 128,), jnp.int32),
        pltpu.VMEM((_RCPX + 128,), jnp.int32),
        pltpu.VMEM((_RCPX + 128,), jnp.int32),
        pltpu.VMEM((_RCPX + 128,), jnp.float32),
        pltpu.VMEM((_RCPX + 128,), jnp.float32),
        pltpu.VMEM((_RCPX + 128,), jnp.float32),
        pltpu.VMEM((2, 3, _RC, _W), jnp.float32),
        pltpu.VMEM((2, _RC, _W), jnp.int32),
        pltpu.SemaphoreType.DMA,
        pltpu.SemaphoreType.DMA,
        pltpu.SemaphoreType.DMA,
        pltpu.SemaphoreType.DMA,
        pltpu.SemaphoreType.DMA,
        pltpu.SemaphoreType.DMA,
    ],
    compiler_params=pltpu.CompilerParams(needs_layout_passes=False),
)
def _raster(vt, ct, vtl, ctl, img, tri, dep, CS, *scratch):
    _raster_body(vt, ct, vtl, ctl, img, tri, dep, CS, *scratch)


def kernel(face_vertices, face_colors, return_buffers):
    vt = jnp.transpose(face_vertices, (2, 3, 0, 1))
    ct = jnp.transpose(face_colors, (2, 3, 0, 1))
    vtail = vt[:, :, :, _FA:]
    fi = jnp.arange(_FT, _CH, dtype=jnp.float32)
    xp = jnp.broadcast_to(((fi % 512.0) + 0.5) / 512.0, (3, _B, _CH - _FT))
    yp = jnp.zeros((3, _B, _CH - _FT), jnp.float32) + (0.5 / 512.0)
    zp = jnp.zeros((3, _B, _CH - _FT), jnp.float32) + 2.0 * _BIG
    pad = jnp.stack([xp, yp, zp], axis=1)
    vtl = jnp.concatenate([vtail, pad], axis=3).reshape(-1)
    ctl = ct[:, :, :, _FA:].reshape(-1)
    images, tri, depth, _ = _raster(vt, ct, vtl, ctl)
    flag = jnp.asarray(return_buffers)
    return lax.cond(
        flag,
        lambda: (images, tri, depth),
        lambda: (jnp.zeros_like(images), jnp.full_like(tri, -1),
                 jnp.full_like(depth, _BIG)),
    )
